# baseline (speedup 1.0000x reference)
import jax
import jax.numpy as jnp
from jax import lax
from jax.experimental import pallas as pl
from jax.experimental.pallas import tpu as pltpu
from jax.experimental.pallas import tpu_sc as plsc

_S = 512
_C = 1000
_B = 16384


def _sc_hist_body(seg_hbm, r_hbm, segv, counts):
    wid = lax.axis_index("s") * 2 + lax.axis_index("c")

    @pl.when(wid == 0)
    def _():
        pltpu.sync_copy(seg_hbm, segv)
        lanes = lax.iota(jnp.int32, 16)
        ones16 = jnp.ones((16,), jnp.float32)
        zeros16 = jnp.zeros((16,), jnp.float32)
        tail_mask = lanes >= 8

        @plsc.parallel_loop(0, _S // 16, 1, unroll=8)
        def _(j):
            counts[pl.ds(j * 16, 16)] = zeros16

        @plsc.parallel_loop(0, 62, 1, unroll=4)
        def _(j):
            sv = segv[pl.ds(j * 16, 16)]
            plsc.addupdate_scatter(counts, [sv], ones16)

        sv = segv[pl.ds(984, 16)]
        plsc.addupdate_scatter(counts, [sv], ones16, mask=tail_mask)

        @plsc.parallel_loop(0, _S // 16, 1, unroll=4)
        def _(j):
            v = counts[pl.ds(j * 16, 16)]
            counts[pl.ds(j * 16, 16)] = 1.0 / jnp.maximum(v, 1.0)

        pltpu.sync_copy(counts, r_hbm)


_sc_recip = pl.kernel(
    _sc_hist_body,
    mesh=plsc.VectorSubcoreMesh(core_axis_name="c", subcore_axis_name="s"),
    out_type=jax.ShapeDtypeStruct((_S,), jnp.float32),
    compiler_params=pltpu.CompilerParams(needs_layout_passes=False),
    scratch_types=[
        pltpu.VMEM((_C,), jnp.int32),
        pltpu.VMEM((_S,), jnp.float32),
    ],
)


def _matmul_tc_kernel(seg_ref, r_ref, x_ref, o_ref, w_ref):
    @pl.when(pl.program_id(0) == 0)
    def _():
        seg = seg_ref[:]
        iota = lax.broadcasted_iota(jnp.int32, (_C, _S), 1)
        onehot = (seg == iota).astype(jnp.float32)
        w_ref[:] = (onehot * r_ref[:]).astype(jnp.bfloat16)

    o_ref[:] = lax.dot_general(
        x_ref[:].astype(jnp.bfloat16), w_ref[:],
        (((1,), (0,)), ((), ())),
        preferred_element_type=jnp.float32,
    )


def kernel(outputs, segment_ids, num_segments):
    b, c = outputs.shape
    seg = jnp.minimum(segment_ids.astype(jnp.int32), num_segments - 1)
    recip = _sc_recip(seg).reshape(1, _S)
    blk = 4096
    return pl.pallas_call(
        _matmul_tc_kernel,
        grid=(b // blk,),
        in_specs=[
            pl.BlockSpec((_C, 1), lambda i: (0, 0)),
            pl.BlockSpec((1, _S), lambda i: (0, 0)),
            pl.BlockSpec((blk, c), lambda i: (i, 0)),
        ],
        out_specs=pl.BlockSpec((blk, _S), lambda i: (i, 0)),
        out_shape=jax.ShapeDtypeStruct((b, _S), jnp.float32),
        scratch_shapes=[pltpu.VMEM((_C, _S), jnp.bfloat16)],
        compiler_params=pltpu.CompilerParams(
            dimension_semantics=("arbitrary",),
        ),
    )(seg.reshape(_C, 1), recip, outputs)


# speedup vs baseline: 1.0028x; 1.0028x over previous
import jax
import jax.numpy as jnp
from jax import lax
from jax.experimental import pallas as pl
from jax.experimental.pallas import tpu as pltpu
from jax.experimental.pallas import tpu_sc as plsc

_S = 512
_C = 1000
_B = 16384


def _sc_hist_body(seg_hbm, r_hbm, segv, counts):
    wid = lax.axis_index("s") * 2 + lax.axis_index("c")

    @pl.when(wid == 0)
    def _():
        pltpu.sync_copy(seg_hbm, segv)
        lanes = lax.iota(jnp.int32, 16)
        ones16 = jnp.ones((16,), jnp.float32)
        zeros16 = jnp.zeros((16,), jnp.float32)
        tail_mask = lanes >= 8

        @plsc.parallel_loop(0, _S // 16, 1, unroll=8)
        def _(j):
            counts[pl.ds(j * 16, 16)] = zeros16

        @plsc.parallel_loop(0, 62, 1, unroll=4)
        def _(j):
            sv = segv[pl.ds(j * 16, 16)]
            plsc.addupdate_scatter(counts, [sv], ones16)

        sv = segv[pl.ds(984, 16)]
        plsc.addupdate_scatter(counts, [sv], ones16, mask=tail_mask)

        @plsc.parallel_loop(0, _S // 16, 1, unroll=4)
        def _(j):
            v = counts[pl.ds(j * 16, 16)]
            counts[pl.ds(j * 16, 16)] = 1.0 / jnp.maximum(v, 1.0)

        pltpu.sync_copy(counts, r_hbm)


_sc_recip = pl.kernel(
    _sc_hist_body,
    mesh=plsc.VectorSubcoreMesh(core_axis_name="c", subcore_axis_name="s"),
    out_type=jax.ShapeDtypeStruct((_S,), jnp.float32),
    compiler_params=pltpu.CompilerParams(needs_layout_passes=False),
    scratch_types=[
        pltpu.VMEM((_C,), jnp.int32),
        pltpu.VMEM((_S,), jnp.float32),
    ],
)


def _matmul_tc_kernel(seg_ref, r_ref, x_ref, o_ref):
    seg = seg_ref[:]
    iota = lax.broadcasted_iota(jnp.int32, (_C, _S), 1)
    onehot = (seg == iota).astype(jnp.float32)
    w = (onehot * r_ref[:]).astype(jnp.bfloat16)
    o_ref[:] = lax.dot_general(
        x_ref[:].astype(jnp.bfloat16), w,
        (((1,), (0,)), ((), ())),
        preferred_element_type=jnp.float32,
    )


def kernel(outputs, segment_ids, num_segments):
    b, c = outputs.shape
    seg = jnp.minimum(segment_ids.astype(jnp.int32), num_segments - 1)
    recip = _sc_recip(seg).reshape(1, _S)
    blk = 4096
    return pl.pallas_call(
        _matmul_tc_kernel,
        grid=(b // blk,),
        in_specs=[
            pl.BlockSpec((_C, 1), lambda i: (0, 0)),
            pl.BlockSpec((1, _S), lambda i: (0, 0)),
            pl.BlockSpec((blk, c), lambda i: (i, 0)),
        ],
        out_specs=pl.BlockSpec((blk, _S), lambda i: (i, 0)),
        out_shape=jax.ShapeDtypeStruct((b, _S), jnp.float32),
        compiler_params=pltpu.CompilerParams(
            dimension_semantics=("arbitrary",),
        ),
    )(seg.reshape(_C, 1), recip, outputs)
